# baseline (speedup 1.0000x reference)
import functools

import jax
import jax.numpy as jnp
from jax import lax
from jax.experimental import pallas as pl
from jax.experimental.pallas import tpu as pltpu
from jax.experimental.pallas import tpu_sc as plsc

_B = 16384
_F = 26
_V = 100000
_D = 32

_NC = 2
_NS = 16
_NW = _NC * _NS
_L = 16

_VCROP = 99968
_CH = 896
_CPF = 112
_CH_TAIL = _VCROP - 111 * _CH
_UNITS = _F * _CPF
_UPT = _UNITS // _NW
_QPU = _CH * _D // 128
_QPU_TAIL = _CH_TAIL * _D // 128
_SRMAIN = _F * _VCROP * _D // 128
_TAILROWS = _F * (_V - _VCROP) * _D // 128
_SR = _SRMAIN + _TAILROWS

_mesh = plsc.VectorSubcoreMesh(
    core_axis_name="c", subcore_axis_name="s", num_cores=_NC, num_subcores=_NS
)


@functools.partial(
    pl.kernel,
    mesh=_mesh,
    out_type=jax.ShapeDtypeStruct((_SR, 128), jnp.float32),
    scratch_types=(
        [pltpu.VMEM((_CH // 128, _D, 128), jnp.float32) for _ in range(2)]
        + [pltpu.VMEM((_QPU, 128), jnp.float32) for _ in range(2)]
        + [pltpu.SemaphoreType.DMA for _ in range(4)]
    ),
    compiler_params=pltpu.CompilerParams(
        use_tc_tiling_on_sc=True, needs_layout_passes=False
    ),
)
def _sc_transpose(tblT_hbm, tail_hbm, out_hbm, blk0, blk1, ob0, ob1, *sems):
    blks = (blk0, blk1)
    obs = (ob0, ob1)
    isems = sems[0:2]
    osems = sems[2:4]
    tid = lax.axis_index("s") * _NC + lax.axis_index("c")

    def unit(i):
        u = i * _NW + tid
        f = u // _CPF
        c = u - f * _CPF
        return f, c

    _NT = _CH // 128
    _NT_TAIL = _CH_TAIL // 128

    def fire_in(i, b):
        f, c = unit(i)
        v0 = c * _CH

        @pl.when(c < _CPF - 1)
        def _():
            for t in range(_NT):
                pltpu.async_copy(
                    tblT_hbm.at[f, :, pl.ds(v0 + t * 128, 128)],
                    blks[b].at[t],
                    isems[b],
                )

        @pl.when(c == _CPF - 1)
        def _():
            for t in range(_NT_TAIL):
                pltpu.async_copy(
                    tblT_hbm.at[f, :, pl.ds(v0 + t * 128, 128)],
                    blks[b].at[t],
                    isems[b],
                )

    def wait_in(i, b):
        f, c = unit(i)

        @pl.when(c < _CPF - 1)
        def _():
            for t in range(_NT):
                pltpu.make_async_copy(
                    tblT_hbm.at[0, :, pl.ds(0, 128)], blks[b].at[t], isems[b]
                ).wait()

        @pl.when(c == _CPF - 1)
        def _():
            for t in range(_NT_TAIL):
                pltpu.make_async_copy(
                    tblT_hbm.at[0, :, pl.ds(0, 128)], blks[b].at[t], isems[b]
                ).wait()

    def fire_out(i, b):
        f, c = unit(i)
        q0 = f * (_VCROP * _D // 128) + c * _QPU

        @pl.when(c < _CPF - 1)
        def _():
            pltpu.async_copy(obs[b], out_hbm.at[pl.ds(q0, _QPU)], osems[b])

        @pl.when(c == _CPF - 1)
        def _():
            pltpu.async_copy(
                obs[b].at[pl.ds(0, _QPU_TAIL)],
                out_hbm.at[pl.ds(q0, _QPU_TAIL)],
                osems[b],
            )

    def wait_out(i, b):
        f, c = unit(i)

        @pl.when(c < _CPF - 1)
        def _():
            pltpu.make_async_copy(
                obs[b], out_hbm.at[pl.ds(0, _QPU)], osems[b]
            ).wait()

        @pl.when(c == _CPF - 1)
        def _():
            pltpu.make_async_copy(
                obs[b].at[pl.ds(0, _QPU_TAIL)],
                out_hbm.at[pl.ds(0, _QPU_TAIL)],
                osems[b],
            ).wait()

    iota16 = lax.iota(jnp.int32, _L)
    row_lo = iota16
    row_hi = iota16 + _L
    rots = [lax.rem(iota16 + p, _L) for p in range(_L)]
    rowq_offs = [r >> 2 for r in rots]
    colq_los = [((r & 3) << 5) + row_lo for r in rots]

    def transpose_block(b, nt):
        for t in range(nt):
            def chunk(m, carry):
                qbase = t * 32 + 4 * m
                vbase = m * 16
                for h, rowv in ((0, row_lo), (1, row_hi)):
                    vals = [
                        plsc.load_gather(blks[b].at[t], [rowv, rots[p] + vbase])
                        for p in range(_L)
                    ]
                    for p in range(_L):
                        rowq = rowq_offs[p] + qbase
                        colq = colq_los[p] + 16 if h else colq_los[p]
                        plsc.store_scatter(obs[b], [rowq, colq], vals[p])
                return carry

            lax.fori_loop(0, 8, chunk, 0)

    fire_in(0, 0)

    def unit_body(i, carry):
        def with_buf(b):
            @pl.when(lax.rem(i, 2) == b)
            def _():
                wait_in(i, b)

                @pl.when(i + 1 < _UPT)
                def _():
                    fire_in(i + 1, 1 - b)

                @pl.when(i >= 2)
                def _():
                    wait_out(i - 2, b)

                _, c = unit(i)

                @pl.when(c < _CPF - 1)
                def _():
                    transpose_block(b, _NT)

                @pl.when(c == _CPF - 1)
                def _():
                    transpose_block(b, _NT_TAIL)

                fire_out(i, b)

        with_buf(0)
        with_buf(1)
        return carry

    lax.fori_loop(0, _UPT, unit_body, 0)
    wait_out(_UPT - 2, (_UPT - 2) % 2)
    wait_out(_UPT - 1, (_UPT - 1) % 2)

    @pl.when(tid == 0)
    def _():
        pltpu.sync_copy(tail_hbm, ob0.at[pl.ds(0, _TAILROWS)])
        pltpu.sync_copy(
            ob0.at[pl.ds(0, _TAILROWS)], out_hbm.at[pl.ds(_SRMAIN, _TAILROWS)]
        )


_BPW = _B // _NW
_EPG = 4
_IPG = _EPG * _F
_GPW = _BPW // _EPG
_NBUF = 16
_NIT = _GPW // _NBUF


@functools.partial(
    pl.kernel,
    mesh=_mesh,
    out_type=jax.ShapeDtypeStruct((_B, _D), jnp.float32),
    scratch_types=(
        [pltpu.VMEM((_GPW, _IPG), jnp.int32)]
        + [pltpu.VMEM((_BPW, _D), jnp.float32)]
        + [pltpu.VMEM((_IPG, _D), jnp.float32) for _ in range(_NBUF)]
        + [pltpu.SemaphoreType.DMA for _ in range(_NBUF)]
    ),
    compiler_params=pltpu.CompilerParams(use_tc_tiling_on_sc=False),
)
def _sc_embed_sum(flat_hbm, tbl_hbm, out_hbm, idx_v, out_v, *rest):
    rows = rest[:_NBUF]
    sems = rest[_NBUF:]
    wid = lax.axis_index("s") * _NC + lax.axis_index("c")

    pltpu.sync_copy(flat_hbm.at[pl.ds(wid * _GPW, _GPW)], idx_v)

    def fire(g, b):
        pltpu.async_copy(tbl_hbm.at[idx_v.at[g]], rows[b], sems[b])

    for b in range(_NBUF):
        fire(b, b)

    def body(i, carry):
        for b in range(_NBUF):
            g = i * _NBUF + b
            pltpu.make_async_copy(tbl_hbm.at[idx_v.at[0]], rows[b], sems[b]).wait()
            for e in range(_EPG):
                r0 = e * _F
                a0 = rows[b][r0, pl.ds(0, _L)]
                a1 = rows[b][r0, pl.ds(_L, _L)]
                for k in range(1, _F):
                    a0 = a0 + rows[b][r0 + k, pl.ds(0, _L)]
                    a1 = a1 + rows[b][r0 + k, pl.ds(_L, _L)]
                orow = g * _EPG + e
                out_v[orow, pl.ds(0, _L)] = a0
                out_v[orow, pl.ds(_L, _L)] = a1

            @pl.when(i < _NIT - 1)
            def _():
                fire(g + _NBUF, b)

        return carry

    lax.fori_loop(0, _NIT, body, 0)
    pltpu.sync_copy(out_v, out_hbm.at[pl.ds(wid * _BPW, _BPW)])


def kernel(inputs, tables):
    tbl_t = tables.transpose(0, 2, 1)
    tail = tables[:, _VCROP:, :].reshape(_TAILROWS, 128)
    tbl128 = _sc_transpose(tbl_t, tail)

    idx = inputs.astype(jnp.int32)
    f_off = jnp.arange(_F, dtype=jnp.int32)[None, :]
    flat_main = f_off * _VCROP + idx
    flat_tail = _SRMAIN * 4 + f_off * (_V - _VCROP) + (idx - _VCROP)
    flat = jnp.where(idx < _VCROP, flat_main, flat_tail)
    flat2d = flat.reshape(_B * _F // _IPG, _IPG)

    tbl = tbl128.reshape(_SR * 4, _D)
    return _sc_embed_sum(flat2d, tbl)


# speedup vs baseline: 1.0339x; 1.0339x over previous
import functools

import jax
import jax.numpy as jnp
from jax import lax
from jax.experimental import pallas as pl
from jax.experimental.pallas import tpu as pltpu
from jax.experimental.pallas import tpu_sc as plsc

_B = 16384
_F = 26
_V = 100000
_D = 32

_NC = 2
_NS = 16
_NW = _NC * _NS
_L = 16

_VCROP = 99968
_CH = 896
_CPF = 112
_CH_TAIL = _VCROP - 111 * _CH
_UNITS = _F * _CPF
_UPT = _UNITS // _NW
_QPU = _CH * _D // 128
_QPU_TAIL = _CH_TAIL * _D // 128
_SRMAIN = _F * _VCROP * _D // 128
_TAILROWS = _F * (_V - _VCROP) * _D // 128
_SR = _SRMAIN + _TAILROWS

_mesh = plsc.VectorSubcoreMesh(
    core_axis_name="c", subcore_axis_name="s", num_cores=_NC, num_subcores=_NS
)


@functools.partial(
    pl.kernel,
    mesh=_mesh,
    out_type=jax.ShapeDtypeStruct((_SR, 128), jnp.float32),
    scratch_types=(
        [pltpu.VMEM((_CH // 128, _D, 128), jnp.float32) for _ in range(2)]
        + [pltpu.VMEM((_QPU, 128), jnp.float32) for _ in range(2)]
        + [pltpu.SemaphoreType.DMA for _ in range(4)]
    ),
    compiler_params=pltpu.CompilerParams(
        use_tc_tiling_on_sc=True, needs_layout_passes=False
    ),
)
def _sc_transpose(tblT_hbm, tail_hbm, out_hbm, blk0, blk1, ob0, ob1, *sems):
    blks = (blk0, blk1)
    obs = (ob0, ob1)
    isems = sems[0:2]
    osems = sems[2:4]
    tid = lax.axis_index("s") * _NC + lax.axis_index("c")

    def unit(i):
        u = i * _NW + tid
        f = u // _CPF
        c = u - f * _CPF
        return f, c

    _NT = _CH // 128
    _NT_TAIL = _CH_TAIL // 128

    def fire_in(i, b):
        f, c = unit(i)
        v0 = c * _CH

        @pl.when(c < _CPF - 1)
        def _():
            for t in range(_NT):
                pltpu.async_copy(
                    tblT_hbm.at[f, :, pl.ds(v0 + t * 128, 128)],
                    blks[b].at[t],
                    isems[b],
                )

        @pl.when(c == _CPF - 1)
        def _():
            for t in range(_NT_TAIL):
                pltpu.async_copy(
                    tblT_hbm.at[f, :, pl.ds(v0 + t * 128, 128)],
                    blks[b].at[t],
                    isems[b],
                )

    def wait_in(i, b):
        f, c = unit(i)

        @pl.when(c < _CPF - 1)
        def _():
            for t in range(_NT):
                pltpu.make_async_copy(
                    tblT_hbm.at[0, :, pl.ds(0, 128)], blks[b].at[t], isems[b]
                ).wait()

        @pl.when(c == _CPF - 1)
        def _():
            for t in range(_NT_TAIL):
                pltpu.make_async_copy(
                    tblT_hbm.at[0, :, pl.ds(0, 128)], blks[b].at[t], isems[b]
                ).wait()

    def fire_out(i, b):
        f, c = unit(i)
        q0 = f * (_VCROP * _D // 128) + c * _QPU

        @pl.when(c < _CPF - 1)
        def _():
            pltpu.async_copy(obs[b], out_hbm.at[pl.ds(q0, _QPU)], osems[b])

        @pl.when(c == _CPF - 1)
        def _():
            pltpu.async_copy(
                obs[b].at[pl.ds(0, _QPU_TAIL)],
                out_hbm.at[pl.ds(q0, _QPU_TAIL)],
                osems[b],
            )

    def wait_out(i, b):
        f, c = unit(i)

        @pl.when(c < _CPF - 1)
        def _():
            pltpu.make_async_copy(
                obs[b], out_hbm.at[pl.ds(0, _QPU)], osems[b]
            ).wait()

        @pl.when(c == _CPF - 1)
        def _():
            pltpu.make_async_copy(
                obs[b].at[pl.ds(0, _QPU_TAIL)],
                out_hbm.at[pl.ds(0, _QPU_TAIL)],
                osems[b],
            ).wait()

    iota16 = lax.iota(jnp.int32, _L)
    row_lo = iota16
    row_hi = iota16 + _L
    rots = [lax.rem(iota16 + p, _L) for p in range(_L)]
    rowq_offs = [r >> 2 for r in rots]
    colq_los = [((r & 3) << 5) + row_lo for r in rots]

    def transpose_block(b, nt):
        for t in range(nt):
            def chunk(m, carry):
                qbase = t * 32 + 4 * m
                vbase = m * 16
                for h, rowv in ((0, row_lo), (1, row_hi)):
                    vals = [
                        plsc.load_gather(blks[b].at[t], [rowv, rots[p] + vbase])
                        for p in range(_L)
                    ]
                    for p in range(_L):
                        rowq = rowq_offs[p] + qbase
                        colq = colq_los[p] + 16 if h else colq_los[p]
                        plsc.store_scatter(obs[b], [rowq, colq], vals[p])
                return carry

            lax.fori_loop(0, 8, chunk, 0)

    fire_in(0, 0)

    def unit_body(i, carry):
        def with_buf(b):
            @pl.when(lax.rem(i, 2) == b)
            def _():
                wait_in(i, b)

                @pl.when(i + 1 < _UPT)
                def _():
                    fire_in(i + 1, 1 - b)

                @pl.when(i >= 2)
                def _():
                    wait_out(i - 2, b)

                _, c = unit(i)

                @pl.when(c < _CPF - 1)
                def _():
                    transpose_block(b, _NT)

                @pl.when(c == _CPF - 1)
                def _():
                    transpose_block(b, _NT_TAIL)

                fire_out(i, b)

        with_buf(0)
        with_buf(1)
        return carry

    lax.fori_loop(0, _UPT, unit_body, 0)
    wait_out(_UPT - 2, (_UPT - 2) % 2)
    wait_out(_UPT - 1, (_UPT - 1) % 2)

    @pl.when(tid == 0)
    def _():
        pltpu.sync_copy(tail_hbm, ob0.at[pl.ds(0, _TAILROWS)])
        pltpu.sync_copy(
            ob0.at[pl.ds(0, _TAILROWS)], out_hbm.at[pl.ds(_SRMAIN, _TAILROWS)]
        )


_BPW = _B // _NW
_EPG = 4
_IPG = _EPG * _F
_GPW = _BPW // _EPG
_NBUF = 8
_NIT = _GPW // _NBUF


@functools.partial(
    pl.kernel,
    mesh=_mesh,
    out_type=jax.ShapeDtypeStruct((_B, _D), jnp.float32),
    scratch_types=(
        [pltpu.VMEM((_GPW, _IPG), jnp.int32)]
        + [pltpu.VMEM((_BPW, _D), jnp.float32)]
        + [pltpu.VMEM((_IPG, _D), jnp.float32) for _ in range(_NBUF)]
        + [pltpu.SemaphoreType.DMA for _ in range(_NBUF)]
    ),
    compiler_params=pltpu.CompilerParams(use_tc_tiling_on_sc=False),
)
def _sc_embed_sum(flat_hbm, tbl_hbm, out_hbm, idx_v, out_v, *rest):
    rows = rest[:_NBUF]
    sems = rest[_NBUF:]
    wid = lax.axis_index("s") * _NC + lax.axis_index("c")

    pltpu.sync_copy(flat_hbm.at[pl.ds(wid * _GPW, _GPW)], idx_v)

    def fire(g, b):
        pltpu.async_copy(tbl_hbm.at[idx_v.at[g]], rows[b], sems[b])

    for b in range(_NBUF):
        fire(b, b)

    def body(i, carry):
        for b in range(_NBUF):
            g = i * _NBUF + b
            pltpu.make_async_copy(tbl_hbm.at[idx_v.at[0]], rows[b], sems[b]).wait()
            for e in range(_EPG):
                r0 = e * _F
                a0 = rows[b][r0, pl.ds(0, _L)]
                a1 = rows[b][r0, pl.ds(_L, _L)]
                for k in range(1, _F):
                    a0 = a0 + rows[b][r0 + k, pl.ds(0, _L)]
                    a1 = a1 + rows[b][r0 + k, pl.ds(_L, _L)]
                orow = g * _EPG + e
                out_v[orow, pl.ds(0, _L)] = a0
                out_v[orow, pl.ds(_L, _L)] = a1

            @pl.when(i < _NIT - 1)
            def _():
                fire(g + _NBUF, b)

        return carry

    lax.fori_loop(0, _NIT, body, 0)
    pltpu.sync_copy(out_v, out_hbm.at[pl.ds(wid * _BPW, _BPW)])


def kernel(inputs, tables):
    tbl_t = tables.transpose(0, 2, 1)
    tail = tables[:, _VCROP:, :].reshape(_TAILROWS, 128)
    tbl128 = _sc_transpose(tbl_t, tail)

    idx = inputs.astype(jnp.int32)
    f_off = jnp.arange(_F, dtype=jnp.int32)[None, :]
    flat_main = f_off * _VCROP + idx
    flat_tail = _SRMAIN * 4 + f_off * (_V - _VCROP) + (idx - _VCROP)
    flat = jnp.where(idx < _VCROP, flat_main, flat_tail)
    flat2d = flat.reshape(_B * _F // _IPG, _IPG)

    tbl = tbl128.reshape(_SR * 4, _D)
    return _sc_embed_sum(flat2d, tbl)
